# Initial kernel scaffold; baseline (speedup 1.0000x reference)
import jax
import jax.numpy as jnp
from jax.experimental import pallas as pl


def kernel(hidden_states, conf_w, conf_b, gate_w, up_w, down_w, expert_wealth):
    raise NotImplementedError("write your pallas kernel here")


# speedup vs baseline: 1.3424x; 1.3424x over previous
import functools

import jax
import jax.numpy as jnp
from jax import lax
from jax.experimental import pallas as pl
from jax.experimental.pallas import tpu as pltpu
from jax.experimental.pallas import tpu_sc as plsc

B, S, D = 2, 4096, 768
E, TOPK, INTER = 64, 2, 1024
CAP = 256
N = B * S
CAP2 = 2 * CAP
NSLOT = E * CAP2
EPAD = E + 1
NROWS = EPAD * CAP2
TRASH = 256
TBLK = 512
IBLK = 512
NC, NS = 2, 16
NW = NC * NS
NPW = N // NW
EPW = E // NW

def _sc_mesh():
    return plsc.VectorSubcoreMesh(
        core_axis_name="c", subcore_axis_name="s", num_cores=NC, num_subcores=NS)


def _routing_body(x_ref, cw_ref, cb_ref, ew_ref, s0_ref, s1_ref, w0_ref, w1_ref):
    x = x_ref[...]
    cw = cw_ref[...]
    conf = lax.dot_general(x, cw, (((1,), (1,)), ((), ())),
                           preferred_element_type=jnp.float32)
    bids = (conf + cb_ref[...]) * ew_ref[...]
    ii = lax.broadcasted_iota(jnp.int32, (TBLK, E), 1)
    v0 = jnp.max(bids, axis=1, keepdims=True)
    s0 = jnp.min(jnp.where(bids == v0, ii, E), axis=1, keepdims=True)
    masked = jnp.where(ii == s0, jnp.finfo(jnp.float32).min, bids)
    v1 = jnp.max(masked, axis=1, keepdims=True)
    s1 = jnp.min(jnp.where(masked == v1, ii, E), axis=1, keepdims=True)
    t = jnp.exp(v1 - v0)
    den = 1.0 + t
    s0_ref[0, 0, :] = s0[:, 0]
    s1_ref[0, 0, :] = s1[:, 0]
    w0_ref[0, 0, :] = (1.0 / den)[:, 0]
    w1_ref[0, 0, :] = (t / den)[:, 0]


def _routing(flat, conf_w, conf_b, expert_wealth):
    nblk = N // TBLK
    out_i = jax.ShapeDtypeStruct((nblk, 1, TBLK), jnp.int32)
    out_f = jax.ShapeDtypeStruct((nblk, 1, TBLK), jnp.float32)
    s0, s1, w0, w1 = pl.pallas_call(
        _routing_body,
        grid=(nblk,),
        in_specs=[
            pl.BlockSpec((TBLK, D), lambda i: (i, 0)),
            pl.BlockSpec((E, D), lambda i: (0, 0)),
            pl.BlockSpec((1, E), lambda i: (0, 0)),
            pl.BlockSpec((1, E), lambda i: (0, 0)),
        ],
        out_specs=[
            pl.BlockSpec((1, 1, TBLK), lambda i: (i, 0, 0)),
            pl.BlockSpec((1, 1, TBLK), lambda i: (i, 0, 0)),
            pl.BlockSpec((1, 1, TBLK), lambda i: (i, 0, 0)),
            pl.BlockSpec((1, 1, TBLK), lambda i: (i, 0, 0)),
        ],
        out_shape=[out_i, out_i, out_f, out_f],
    )(flat, conf_w, conf_b.reshape(1, E), expert_wealth.reshape(1, E))
    return (s0.reshape(N), s1.reshape(N), w0.reshape(N), w1.reshape(N))


def _dispatch_body(flat_hbm, sel0_hbm, sel1_hbm, rw0_hbm, rw1_hbm,
                   xg_hbm, ws_hbm, src0_hbm, src1_hbm,
                   sel0_v, sel1_v, rw0_v, rw1_v,
                   tok0_v, tok1_v, tok2_v, tok3_v,
                   wsl_v, idx_v, val_v, gidx0_v, gidx1_v, rows_v,
                   zs_v, sem_s, sem_g0, sem_g1):
    wid = lax.axis_index("s") * NC + lax.axis_index("c")
    sels_v = [sel0_v, sel1_v]
    rws_v = [rw0_v, rw1_v]
    toks_v = [tok0_v, tok1_v, tok2_v, tok3_v]
    gidxs_v = [gidx0_v, gidx1_v]
    pltpu.sync_copy(sel0_hbm, sel0_v)
    pltpu.sync_copy(sel1_hbm, sel1_v)
    pltpu.sync_copy(rw0_hbm, rw0_v)
    pltpu.sync_copy(rw1_hbm, rw1_v)

    lanes = lax.iota(jnp.int32, 16)

    zs_v[pl.ds(0, 16)] = jnp.zeros((16,), jnp.float32)
    pltpu.sync_copy(zs_v.at[pl.ds(0, 16)],
                    ws_hbm.at[pl.ds(NSLOT + wid * 16, 16)])

    cnts = [None] * 4
    for k in range(2):
        def chunk_body(c, offs, _k=k):
            s = sels_v[_k][pl.ds(c * 16, 16)]
            toks = lanes + c * 16
            new = []
            ones = jnp.ones((16,), jnp.int32)
            for el in range(EPW):
                e = wid * EPW + el
                m = s == e
                j = el * 2 + _k
                rank = plsc.cumsum(ones, mask=m)
                pos = jnp.where(m, offs[el] + rank - 1, N + lanes)
                plsc.store_scatter(toks_v[j], [pos], toks)
                new.append(offs[el] + jnp.sum(m.astype(jnp.int32)))
            return tuple(new)
        offs = lax.fori_loop(0, N // 16, chunk_body,
                             tuple(jnp.int32(0) for _ in range(EPW)))
        for el in range(EPW):
            cnts[el * 2 + k] = offs[el]

    for j in range(2 * EPW):
        el, k = j // 2, j % 2
        e = wid * EPW + el
        cnt = cnts[j]
        base = (e * 2 + k) * CAP

        for q in range(CAP // 16):
            lane = lanes + q * 16
            toks = jnp.where(lane < cnt, toks_v[j][pl.ds(q * 16, 16)], 0)
            w = plsc.load_gather(rws_v[k], [toks])
            wsl_v[pl.ds(q * 16, 16)] = jnp.where(lane < cnt, w, 0.0)
        pltpu.sync_copy(wsl_v, ws_hbm.at[pl.ds(base, CAP)])

        src_hbm = src0_hbm if k == 0 else src1_hbm
        def s_chunk(c, _, _cnt=cnt, _base=base, _j=j, _src=src_hbm):
            @pl.when(c * 128 < _cnt)
            def _():
                for q in range(8):
                    p0 = c * 128 + q * 16
                    lane = lanes + p0
                    toks = toks_v[_j][pl.ds(p0, 16)]
                    valid = lane < _cnt
                    idx_v[0, pl.ds(q * 16, 16)] = jnp.where(
                        valid, toks, N + (lane & (TRASH - 1)))
                    val_v[pl.ds(q * 16, 16)] = jnp.where(
                        lane < jnp.minimum(_cnt, CAP), _base + lane,
                        NSLOT + (lane & (CAP2 - 1)))
                pltpu.async_copy(val_v, _src.at[idx_v.at[0]], sem_s).wait()
            return 0
        lax.fori_loop(0, N // 128, s_chunk, 0)

        def g_chunk(c, _, _cnt=cnt, _base=base, _j=j):
            @pl.when(c * 16 < _cnt)
            def _():
                p0 = c * 16
                lane = lanes + p0
                toks = toks_v[_j][pl.ds(p0, 16)]
                gidx0_v[...] = jnp.where(lane < _cnt, toks, lane)
                pltpu.async_copy(flat_hbm.at[gidx0_v], rows_v.at[0],
                                 sem_g0).wait()
                pltpu.sync_copy(rows_v.at[0],
                                xg_hbm.at[pl.ds(_base + p0, 16), :])
            return 0
        lax.fori_loop(0, CAP // 16, g_chunk, 0)


def _dispatch(flat, sel0, sel1, rw0, rw1):
    f = pl.kernel(
        _dispatch_body,
        out_type=[
            jax.ShapeDtypeStruct((NSLOT, D), jnp.float32),
            jax.ShapeDtypeStruct((NROWS,), jnp.float32),
            jax.ShapeDtypeStruct((N + TRASH,), jnp.int32),
            jax.ShapeDtypeStruct((N + TRASH,), jnp.int32),
        ],
        mesh=_sc_mesh(),
        compiler_params=pltpu.CompilerParams(needs_layout_passes=False),
        scratch_types=[
            pltpu.VMEM((N,), jnp.int32),
            pltpu.VMEM((N,), jnp.int32),
            pltpu.VMEM((N,), jnp.float32),
            pltpu.VMEM((N,), jnp.float32),
            pltpu.VMEM((N + 16,), jnp.int32),
            pltpu.VMEM((N + 16,), jnp.int32),
            pltpu.VMEM((N + 16,), jnp.int32),
            pltpu.VMEM((N + 16,), jnp.int32),
            pltpu.VMEM((CAP,), jnp.float32),
            pltpu.VMEM((1, 128), jnp.int32),
            pltpu.VMEM((128,), jnp.int32),
            pltpu.VMEM((16,), jnp.int32),
            pltpu.VMEM((16,), jnp.int32),
            pltpu.VMEM((2, 16, D), jnp.float32),
            pltpu.VMEM((CAP,), jnp.float32),
            pltpu.SemaphoreType.DMA,
            pltpu.SemaphoreType.DMA,
            pltpu.SemaphoreType.DMA,
        ],
    )
    return f(flat, sel0, sel1, rw0, rw1)


def _ffn_body(xg_ref, gw_ref, uw_ref, dw_ref, ws_ref, out_ref, acc_ref):
    i = pl.program_id(1)
    x = xg_ref[0].astype(jnp.bfloat16)
    g = lax.dot(x, gw_ref[0].astype(jnp.bfloat16),
                preferred_element_type=jnp.float32)
    u = lax.dot(x, uw_ref[0].astype(jnp.bfloat16),
                preferred_element_type=jnp.float32)
    h = (g * jax.nn.sigmoid(g) * u).astype(jnp.bfloat16)
    p = lax.dot(h, dw_ref[0].astype(jnp.bfloat16),
                preferred_element_type=jnp.float32)

    @pl.when(i == 0)
    def _():
        acc_ref[...] = p

    @pl.when(i == 1)
    def _():
        out_ref[0] = (acc_ref[...] + p) * ws_ref[0]


def _ffn(xg, gate_w, up_w, down_w, wslot):
    nI = INTER // IBLK
    return pl.pallas_call(
        _ffn_body,
        grid=(EPAD, nI),
        in_specs=[
            pl.BlockSpec((1, CAP2, D), lambda e, i: (jnp.minimum(e, E - 1), 0, 0)),
            pl.BlockSpec((1, D, IBLK), lambda e, i: (jnp.minimum(e, E - 1), 0, i)),
            pl.BlockSpec((1, D, IBLK), lambda e, i: (jnp.minimum(e, E - 1), 0, i)),
            pl.BlockSpec((1, IBLK, D), lambda e, i: (jnp.minimum(e, E - 1), i, 0)),
            pl.BlockSpec((1, CAP2, 1), lambda e, i: (e, 0, 0)),
        ],
        out_specs=pl.BlockSpec((1, CAP2, D), lambda e, i: (e, 0, 0)),
        out_shape=jax.ShapeDtypeStruct((EPAD, CAP2, D), jnp.float32),
        scratch_shapes=[pltpu.VMEM((CAP2, D), jnp.float32)],
    )(xg.reshape(E, CAP2, D), gate_w, up_w, down_w,
      wslot.reshape(EPAD, CAP2, 1))


def _combine_body(eo_hbm, src0_hbm, src1_hbm, out_hbm,
                  s0_v, s1_v, ra0_v, ra1_v, rb0_v, rb1_v, ob0_v, ob1_v,
                  sem_a, sem_b, sem_o):
    wid = lax.axis_index("s") * NC + lax.axis_index("c")
    t0 = wid * NPW
    pltpu.sync_copy(src0_hbm.at[pl.ds(t0, NPW)], s0_v)
    pltpu.sync_copy(src1_hbm.at[pl.ds(t0, NPW)], s1_v)

    nch = NPW // 16

    def chunk(c, _):
        da = pltpu.async_copy(eo_hbm.at[s0_v.at[pl.ds(c * 16, 16)]],
                              ra0_v, sem_a)
        db = pltpu.async_copy(eo_hbm.at[s1_v.at[pl.ds(c * 16, 16)]],
                              rb0_v, sem_b)
        da.wait()
        db.wait()

        def add_body(i, _):
            r = i // (D // 16)
            w = (i % (D // 16)) * 16
            ob0_v[r, pl.ds(w, 16)] = (ra0_v[r, pl.ds(w, 16)]
                                      + rb0_v[r, pl.ds(w, 16)])
            return 0
        lax.fori_loop(0, 16 * (D // 16), add_body, 0)
        pltpu.sync_copy(ob0_v, out_hbm.at[pl.ds(t0 + c * 16, 16), :])
        return 0

    lax.fori_loop(0, nch, chunk, 0)


def _combine(eo, src0, src1):
    f = pl.kernel(
        _combine_body,
        out_type=jax.ShapeDtypeStruct((N, D), jnp.float32),
        mesh=_sc_mesh(),
        compiler_params=pltpu.CompilerParams(needs_layout_passes=False),
        scratch_types=[
            pltpu.VMEM((NPW,), jnp.int32),
            pltpu.VMEM((NPW,), jnp.int32),
            pltpu.VMEM((16, D), jnp.float32),
            pltpu.VMEM((16, D), jnp.float32),
            pltpu.VMEM((16, D), jnp.float32),
            pltpu.VMEM((16, D), jnp.float32),
            pltpu.VMEM((16, D), jnp.float32),
            pltpu.VMEM((16, D), jnp.float32),
            pltpu.SemaphoreType.DMA,
            pltpu.SemaphoreType.DMA,
            pltpu.SemaphoreType.DMA,
        ],
    )
    return f(eo, src0, src1)


def kernel(hidden_states, conf_w, conf_b, gate_w, up_w, down_w, expert_wealth):
    flat = hidden_states.reshape(N, D)
    sel0, sel1, rw0, rw1 = _routing(flat, conf_w, conf_b, expert_wealth)
    xg, wslot, src0, src1 = _dispatch(flat, sel0, sel1, rw0, rw1)
    eo = _ffn(xg, gate_w, up_w, down_w, wslot)
    out = _combine(eo.reshape(NROWS, D), src0, src1)
    return out.reshape(B, S, D)
